# baseline (speedup 1.0000x reference)
import functools

import jax
import jax.numpy as jnp
import numpy as np
from jax import lax
from jax.experimental import pallas as pl
from jax.experimental.pallas import tpu as pltpu
from jax.experimental.pallas import tpu_sc as plsc

H = 128
B = 1024
N_PER = 32
L = 16
PAD = 8
F32 = jnp.float32

NW = 32
IS_TOTAL = B * L
IS_PER_W = IS_TOTAL // NW
CH = 8
NCH = IS_PER_W // CH
ZROW = B * N_PER

RB = 2048
SS = RB // L
TV = 2048


def _dot_t(x, w):
    return lax.dot_general(x, w, (((1,), (1,)), ((), ())),
                           preferred_element_type=F32)


def _pool_sc(nodes_hbm, seqf_hbm, il_hbm, out_hbm,
             seq_v, il_v, idx_v, rows_v, out_v, gsem, osem):
    wid = lax.axis_index("s") * 2 + lax.axis_index("c")
    it0 = wid * IS_PER_W
    f0 = it0 * PAD

    pltpu.sync_copy(seqf_hbm.at[pl.ds(f0, IS_PER_W * PAD)], seq_v)
    pltpu.sync_copy(il_hbm.at[pl.ds(it0, IS_PER_W)],
                    il_v.at[pl.ds(0, IS_PER_W)])

    lane = lax.broadcasted_iota(jnp.int32, (16,), 0)

    def compute_idx(c, p):
        base = c * (CH * PAD)
        for v in range(CH * PAD // 16):
            pos = base + v * 16 + lane
            s = seq_v[pl.ds(base + v * 16, 16)]
            sess = (it0 + (pos >> 3)) >> 4
            g = sess * N_PER + s
            g = jnp.where(s < N_PER, g, ZROW)
            idx_v[p, pl.ds(v * 16, 16)] = g

    def start_gather(c, p):
        return pltpu.async_copy(nodes_hbm.at[idx_v.at[p]], rows_v.at[p],
                                gsem.at[p])

    def accum(c, p):
        ilv16 = il_v[pl.ds(c * CH, 16)].astype(F32)
        inv16 = 1.0 / ilv16
        for t in range(CH):
            inv = jnp.full((16,), inv16[t], F32)
            for hc in range(H // 16):
                acc = rows_v[p, t * PAD, pl.ds(hc * 16, 16)]
                for j in range(1, PAD):
                    acc = acc + rows_v[p, t * PAD + j, pl.ds(hc * 16, 16)]
                out_v[p, t, pl.ds(hc * 16, 16)] = acc * inv

    compute_idx(0, 0)
    g0 = start_gather(0, 0)

    def body(c, _):
        for par in range(2):
            @pl.when(c + par < NCH)
            def _():
                pltpu.make_async_copy(nodes_hbm.at[idx_v.at[par]],
                                      rows_v.at[par], gsem.at[par]).wait()

                @pl.when(c + par + 1 < NCH)
                def _():
                    compute_idx(c + par + 1, 1 - par)
                    pltpu.async_copy(nodes_hbm.at[idx_v.at[1 - par]],
                                     rows_v.at[1 - par], gsem.at[1 - par])

                @pl.when(c + par >= 2)
                def _():
                    pltpu.make_async_copy(
                        out_v.at[par],
                        out_hbm.at[pl.ds(it0 + (c + par - 2) * CH, CH)],
                        osem.at[par]).wait()
                accum(c + par, par)
                pltpu.async_copy(
                    out_v.at[par],
                    out_hbm.at[pl.ds(it0 + (c + par) * CH, CH)],
                    osem.at[par])
        return ()

    lax.fori_loop(0, NCH // 2, lambda i, carry: body(i * 2, carry), (),
                  unroll=False)
    for par in range(2):
        pltpu.make_async_copy(
            out_v.at[par],
            out_hbm.at[pl.ds(it0 + (NCH - 2 + par) * CH, CH)],
            osem.at[par]).wait()


def _pool_sessions(node_embedding, sequence, itemset_len):
    nodes_z = jnp.concatenate(
        [node_embedding, jnp.zeros((8, H), F32)], axis=0)
    seqf = sequence.reshape(IS_TOTAL * PAD)
    mesh = plsc.VectorSubcoreMesh(core_axis_name="c", subcore_axis_name="s")
    f = pl.kernel(
        _pool_sc,
        mesh=mesh,
        out_type=jax.ShapeDtypeStruct((IS_TOTAL, H), F32),
        scratch_types=[
            pltpu.VMEM((IS_PER_W * PAD,), jnp.int32),
            pltpu.VMEM((IS_PER_W + 16,), jnp.int32),
            pltpu.VMEM((2, CH * PAD), jnp.int32),
            pltpu.VMEM((2, CH * PAD, H), F32),
            pltpu.VMEM((2, CH, H), F32),
            pltpu.SemaphoreType.DMA((2,)),
            pltpu.SemaphoreType.DMA((2,)),
        ],
    )
    return f(nodes_z, seqf, itemset_len)


def _cue_sc(e_hbm, cue_hbm, out_hbm, idx_v, rows_v, sem):
    wid = lax.axis_index("s") * 2 + lax.axis_index("c")
    base = wid * (B // NW)
    pltpu.sync_copy(cue_hbm.at[pl.ds(base, B // NW)], idx_v)
    pltpu.async_copy(e_hbm.at[idx_v], rows_v, sem).wait()
    pltpu.sync_copy(rows_v, out_hbm.at[pl.ds(base, B // NW)])


def _gather_cue(embedding_table_weight, cue):
    mesh = plsc.VectorSubcoreMesh(core_axis_name="c", subcore_axis_name="s")
    f = pl.kernel(
        _cue_sc,
        mesh=mesh,
        out_type=jax.ShapeDtypeStruct((B, H), F32),
        scratch_types=[
            pltpu.VMEM((B // NW,), jnp.int32),
            pltpu.VMEM((B // NW, H), F32),
            pltpu.SemaphoreType.DMA,
        ],
    )
    return f(embedding_table_weight, cue)


def _attn_body(sess_ref, ec_ref, sel_ref, rep_ref, w1_ref, w2_ref, b12_ref,
               q_ref, qb_ref, w3a_ref, w3b_ref, b3_ref, sh_ref, y_ref):
    sess = sess_ref[...]
    v_n = lax.dot_general(sel_ref[...], sess, (((1,), (0,)), ((), ())),
                          preferred_element_type=F32)
    v_n_rep = lax.dot_general(rep_ref[...], v_n, (((1,), (0,)), ((), ())),
                              preferred_element_type=F32)
    a = jax.nn.sigmoid(_dot_t(v_n_rep, w1_ref[...]) + _dot_t(sess, w2_ref[...])
                       + b12_ref[...])
    alpha = jnp.sum(a * q_ref[...], axis=1, keepdims=True) + qb_ref[...]
    s_g = lax.dot_general(rep_ref[...], alpha * sess, (((0,), (0,)), ((), ())),
                          preferred_element_type=F32)
    s_h = (_dot_t(v_n, w3a_ref[...]) + _dot_t(s_g, w3b_ref[...])
           + b3_ref[...])
    sh_ref[...] = s_h
    y_ref[...] = jnp.sum(s_h * ec_ref[...], axis=1, keepdims=True)


def _attention(session, ec, W1_w, W2_w, b12, q_w, qb2, w3a, w3b, b3):
    r = np.arange(RB)
    sel = jnp.asarray((r[None, :] == (np.arange(SS) * L + L - 1)[:, None])
                      .astype(np.float32))
    rep = jnp.asarray((r[:, None] // L == np.arange(SS)[None, :])
                      .astype(np.float32))
    n_blocks = IS_TOTAL // RB
    return pl.pallas_call(
        _attn_body,
        grid=(n_blocks,),
        in_specs=[
            pl.BlockSpec((RB, H), lambda i: (i, 0)),
            pl.BlockSpec((SS, H), lambda i: (i, 0)),
            pl.BlockSpec((SS, RB), lambda i: (0, 0)),
            pl.BlockSpec((RB, SS), lambda i: (0, 0)),
            pl.BlockSpec((H, H), lambda i: (0, 0)),
            pl.BlockSpec((H, H), lambda i: (0, 0)),
            pl.BlockSpec((1, H), lambda i: (0, 0)),
            pl.BlockSpec((1, H), lambda i: (0, 0)),
            pl.BlockSpec((1, 1), lambda i: (0, 0)),
            pl.BlockSpec((H, H), lambda i: (0, 0)),
            pl.BlockSpec((H, H), lambda i: (0, 0)),
            pl.BlockSpec((1, H), lambda i: (0, 0)),
        ],
        out_specs=[
            pl.BlockSpec((SS, H), lambda i: (i, 0)),
            pl.BlockSpec((SS, 1), lambda i: (i, 0)),
        ],
        out_shape=[
            jax.ShapeDtypeStruct((B, H), F32),
            jax.ShapeDtypeStruct((B, 1), F32),
        ],
    )(session, ec, sel, rep, W1_w, W2_w, b12, q_w, qb2, w3a, w3b, b3)


def _stage_b_body(sh_ref, e_ref, out_ref):
    out_ref[...] = _dot_t(sh_ref[...], e_ref[...])


def _scores(s_h, embedding_table_weight):
    vocab = embedding_table_weight.shape[0]
    n_vtiles = pl.cdiv(vocab, TV)
    return pl.pallas_call(
        _stage_b_body,
        grid=(n_vtiles,),
        in_specs=[
            pl.BlockSpec((B, H), lambda k: (0, 0)),
            pl.BlockSpec((TV, H), lambda k: (k, 0)),
        ],
        out_specs=pl.BlockSpec((B, TV), lambda k: (0, k)),
        out_shape=jax.ShapeDtypeStruct((B, vocab), F32),
    )(s_h, embedding_table_weight)


@jax.jit
def kernel(node_embedding, embedding_table_weight, batch, sequence, itemset_len,
           sequence_len, cue, W1_w, W1_b, W2_w, W2_b, q_w, q_b, W3_w, W3_b):
    del batch, sequence_len
    b12 = (W1_b + W2_b).reshape(1, H)
    qb2 = q_b.reshape(1, 1)
    w3a = W3_w[:, :H]
    w3b = W3_w[:, H:]
    b3 = W3_b.reshape(1, H)

    session = _pool_sessions(node_embedding, sequence, itemset_len)
    ec = _gather_cue(embedding_table_weight, cue)
    s_h, y2 = _attention(session, ec, W1_w, W2_w, b12, q_w, qb2,
                         w3a, w3b, b3)
    all_scores = _scores(s_h, embedding_table_weight)
    return (y2.reshape(B), all_scores)
